# Initial kernel scaffold; baseline (speedup 1.0000x reference)
import jax
import jax.numpy as jnp
from jax.experimental import pallas as pl


def kernel(x, edge_index, c1_Wk, c1_bk, c1_Wq, c1_bq, c1_Wv, c1_bv, c1_Ws, c1_b, l1_W, l1_b, c2_Wk, c2_bk, c2_Wq, c2_bq, c2_Wv, c2_bv, c2_Ws, c2_b, l2_W, l2_b):
    raise NotImplementedError("write your pallas kernel here")


# speedup vs baseline: 1.3716x; 1.3716x over previous
import functools

import jax
import jax.numpy as jnp
from jax import lax
from jax.experimental import pallas as pl
from jax.experimental.pallas import tpu as pltpu
from jax.experimental.pallas import tpu_sc as plsc

N = 10000
E = 320000
H = 128

NC = 2
NS = 16
NW = NC * NS
EPT = E // NW
C = 40
SUP = 50
NSUP = EPT // (C * SUP)
ROWCHUNK = 200
NCHUNK = N // ROWCHUNK


_MM_B = 1000


def _qkvs_body(x_ref, w_ref, b_ref, k_ref, qv_ref, s_ref):
    acc = jnp.dot(x_ref[...], w_ref[...], preferred_element_type=jnp.float32)
    acc = acc + b_ref[...]
    k_ref[...] = acc[:, :H]
    qv_ref[...] = acc[:, H:3 * H]
    s_ref[...] = acc[:, 3 * H:]


def _qkvs(x, wcat, bcat):
    grid = (N // _MM_B,)
    return pl.pallas_call(
        _qkvs_body,
        grid=grid,
        in_specs=[
            pl.BlockSpec((_MM_B, H), lambda i: (i, 0)),
            pl.BlockSpec((H, 4 * H), lambda i: (0, 0)),
            pl.BlockSpec((4 * H,), lambda i: (0,)),
        ],
        out_specs=[
            pl.BlockSpec((_MM_B, H), lambda i: (i, 0)),
            pl.BlockSpec((_MM_B, 2 * H), lambda i: (i, 0)),
            pl.BlockSpec((_MM_B, H), lambda i: (i, 0)),
        ],
        out_shape=[
            jax.ShapeDtypeStruct((N, H), jnp.float32),
            jax.ShapeDtypeStruct((N, 2 * H), jnp.float32),
            jax.ShapeDtypeStruct((N, H), jnp.float32),
        ],
    )(x, wcat, bcat)


def _combine_body(a_ref, s_ref, w_ref, b_ref, o_ref):
    t = a_ref[0] + a_ref[1] + s_ref[...]
    acc = jnp.dot(t, w_ref[...], preferred_element_type=jnp.float32)
    o_ref[...] = jnp.maximum(acc + b_ref[...], 0.0)


def _combine(agg2, skip, w, b):
    grid = (N // _MM_B,)
    return pl.pallas_call(
        _combine_body,
        grid=grid,
        in_specs=[
            pl.BlockSpec((NC, _MM_B, H), lambda i: (0, i, 0)),
            pl.BlockSpec((_MM_B, H), lambda i: (i, 0)),
            pl.BlockSpec((H, H), lambda i: (0, 0)),
            pl.BlockSpec((H,), lambda i: (0,)),
        ],
        out_specs=pl.BlockSpec((_MM_B, H), lambda i: (i, 0)),
        out_shape=jax.ShapeDtypeStruct((N, H), jnp.float32),
    )(agg2, skip, w, b)


def _edge_body(k_hbm, qv_hbm, src_hbm, dst_hbm, zeros_hbm, out_hbm,
               src_v, dst_v, k_v, qv_v, msg_v, agg_sh, sem):
    c = lax.axis_index("c")
    s = lax.axis_index("s")
    wid = c * NS + s

    for r in range((NCHUNK + NS - 1) // NS):
        idx = r * NS + s

        @pl.when(idx < NCHUNK)
        def _():
            pltpu.sync_copy(zeros_hbm.at[pl.ds(idx * ROWCHUNK, ROWCHUNK)],
                            agg_sh.at[pl.ds(idx * ROWCHUNK, ROWCHUNK)])

    plsc.subcore_barrier()

    def _super(sup, carry):
        pltpu.sync_copy(src_hbm.at[wid, sup], src_v)
        pltpu.sync_copy(dst_hbm.at[wid, sup], dst_v)

        def _chunk(j, carry2):
            pltpu.async_copy(k_hbm.at[dst_v.at[j]], k_v, sem).wait()
            pltpu.async_copy(qv_hbm.at[src_v.at[j]], qv_v, sem).wait()

            def _edge(e, cc):
                for t in range(H // 16):
                    vk = k_v[e, pl.ds(t * 16, 16)]
                    vq = qv_v[e, pl.ds(t * 16, 16)]
                    vv = qv_v[e, pl.ds(H + t * 16, 16)]
                    eta = 1.0 / (1.0 + jnp.exp(-(vk + vq)))
                    msg_v[e, pl.ds(t * 16, 16)] = eta * vv
                return cc

            lax.fori_loop(0, C, _edge, 0)
            pltpu.sync_copy(msg_v, agg_sh.at[dst_v.at[j]], add=True)
            return carry2

        lax.fori_loop(0, SUP, _chunk, 0)
        return carry

    lax.fori_loop(0, NSUP, _super, 0)
    plsc.subcore_barrier()

    for r in range((NCHUNK + NS - 1) // NS):
        idx = r * NS + s

        @pl.when(idx < NCHUNK)
        def _():
            pltpu.sync_copy(agg_sh.at[pl.ds(idx * ROWCHUNK, ROWCHUNK)],
                            out_hbm.at[c, pl.ds(idx * ROWCHUNK, ROWCHUNK)])


@functools.partial(
    pl.kernel,
    out_type=jax.ShapeDtypeStruct((NC, N, H), jnp.float32),
    mesh=plsc.VectorSubcoreMesh(
        core_axis_name="c", subcore_axis_name="s",
        num_cores=NC, num_subcores=NS),
    scratch_types=[
        pltpu.VMEM((SUP, C), jnp.int32),
        pltpu.VMEM((SUP, C), jnp.int32),
        pltpu.VMEM((C, H), jnp.float32),
        pltpu.VMEM((C, 2 * H), jnp.float32),
        pltpu.VMEM((C, H), jnp.float32),
        pltpu.VMEM_SHARED((N, H), jnp.float32),
        pltpu.SemaphoreType.DMA,
    ],
)
def _edge_agg(k_hbm, qv_hbm, src_hbm, dst_hbm, zeros_hbm, out_hbm,
              src_v, dst_v, k_v, qv_v, msg_v, agg_sh, sem):
    _edge_body(k_hbm, qv_hbm, src_hbm, dst_hbm, zeros_hbm, out_hbm,
               src_v, dst_v, k_v, qv_v, msg_v, agg_sh, sem)


def _layer(x, src4, dst4, zeros, Wk, bk, Wq, bq, Wv, bv, Ws, b, lW, lb):
    wcat = jnp.concatenate([Wk, Wq, Wv, Ws], axis=1)
    bcat = jnp.concatenate([bk, bq, bv, b], axis=0)
    k, qv, skip = _qkvs(x, wcat, bcat)
    agg2 = _edge_agg(k, qv, src4, dst4, zeros)
    return _combine(agg2, skip, lW, lb)


def kernel(x, edge_index, c1_Wk, c1_bk, c1_Wq, c1_bq, c1_Wv, c1_bv, c1_Ws,
           c1_b, l1_W, l1_b, c2_Wk, c2_bk, c2_Wq, c2_bq, c2_Wv, c2_bv,
           c2_Ws, c2_b, l2_W, l2_b):
    src4 = edge_index[0].reshape(NW, NSUP, SUP, C)
    dst4 = edge_index[1].reshape(NW, NSUP, SUP, C)
    zeros = jnp.zeros((N, H), jnp.float32)
    h = _layer(x, src4, dst4, zeros, c1_Wk, c1_bk, c1_Wq, c1_bq, c1_Wv,
               c1_bv, c1_Ws, c1_b, l1_W, l1_b)
    h = _layer(h, src4, dst4, zeros, c2_Wk, c2_bk, c2_Wq, c2_bq, c2_Wv,
               c2_bv, c2_Ws, c2_b, l2_W, l2_b)
    return h


# speedup vs baseline: 1.7054x; 1.2434x over previous
import functools

import jax
import jax.numpy as jnp
from jax import lax
from jax.experimental import pallas as pl
from jax.experimental.pallas import tpu as pltpu
from jax.experimental.pallas import tpu_sc as plsc

N = 10000
E = 320000
H = 128

NC = 2
NS = 16
NW = NC * NS
EPT = E // NW
C = 40
SUP = 50
NSUP = EPT // (C * SUP)
ROWCHUNK = 200
NCHUNK = N // ROWCHUNK


_MM_B = 1000


def _qkvs_body(x_ref, w_ref, b_ref, k_ref, qv_ref, s_ref):
    acc = jnp.dot(x_ref[...], w_ref[...], preferred_element_type=jnp.float32)
    acc = acc + b_ref[...]
    k_ref[...] = acc[:, :H]
    qv_ref[...] = acc[:, H:3 * H]
    s_ref[...] = acc[:, 3 * H:]


def _qkvs(x, wcat, bcat):
    grid = (N // _MM_B,)
    return pl.pallas_call(
        _qkvs_body,
        grid=grid,
        in_specs=[
            pl.BlockSpec((_MM_B, H), lambda i: (i, 0)),
            pl.BlockSpec((H, 4 * H), lambda i: (0, 0)),
            pl.BlockSpec((4 * H,), lambda i: (0,)),
        ],
        out_specs=[
            pl.BlockSpec((_MM_B, H), lambda i: (i, 0)),
            pl.BlockSpec((_MM_B, 2 * H), lambda i: (i, 0)),
            pl.BlockSpec((_MM_B, H), lambda i: (i, 0)),
        ],
        out_shape=[
            jax.ShapeDtypeStruct((N, H), jnp.float32),
            jax.ShapeDtypeStruct((N, 2 * H), jnp.float32),
            jax.ShapeDtypeStruct((N, H), jnp.float32),
        ],
    )(x, wcat, bcat)


def _combine_body(a_ref, s_ref, w_ref, b_ref, o_ref):
    t = a_ref[0] + a_ref[1] + s_ref[...]
    acc = jnp.dot(t, w_ref[...], preferred_element_type=jnp.float32)
    o_ref[...] = jnp.maximum(acc + b_ref[...], 0.0)


def _combine(agg2, skip, w, b):
    grid = (N // _MM_B,)
    return pl.pallas_call(
        _combine_body,
        grid=grid,
        in_specs=[
            pl.BlockSpec((NC, _MM_B, H), lambda i: (0, i, 0)),
            pl.BlockSpec((_MM_B, H), lambda i: (i, 0)),
            pl.BlockSpec((H, H), lambda i: (0, 0)),
            pl.BlockSpec((H,), lambda i: (0,)),
        ],
        out_specs=pl.BlockSpec((_MM_B, H), lambda i: (i, 0)),
        out_shape=jax.ShapeDtypeStruct((N, H), jnp.float32),
    )(agg2, skip, w, b)


def _edge_body(k_hbm, qv_hbm, src_hbm, dst_hbm, zeros_hbm, out_hbm,
               src_v, dst_v, k_v0, qv_v0, k_v1, qv_v1, msg_v, agg_sh,
               sem0, sem1):
    c = lax.axis_index("c")
    s = lax.axis_index("s")
    wid = c * NS + s

    for r in range((NCHUNK + NS - 1) // NS):
        idx = r * NS + s

        @pl.when(idx < NCHUNK)
        def _():
            pltpu.sync_copy(zeros_hbm.at[pl.ds(idx * ROWCHUNK, ROWCHUNK)],
                            agg_sh.at[pl.ds(idx * ROWCHUNK, ROWCHUNK)])

    plsc.subcore_barrier()

    def _start(j, kbuf, qbuf, sem):
        pltpu.async_copy(k_hbm.at[dst_v.at[j]], kbuf, sem)
        pltpu.async_copy(qv_hbm.at[src_v.at[j]], qbuf, sem)

    def _drain(j, kbuf, qbuf, sem):
        pltpu.make_async_copy(k_hbm.at[dst_v.at[j]], kbuf, sem).wait()
        pltpu.make_async_copy(qv_hbm.at[src_v.at[j]], qbuf, sem).wait()

    def _compute_scatter(j, kbuf, qbuf):
        def _edge(e, cc):
            for t in range(H // 16):
                vk = kbuf[e, pl.ds(t * 16, 16)]
                vq = qbuf[e, pl.ds(t * 16, 16)]
                vv = qbuf[e, pl.ds(H + t * 16, 16)]
                eta = 1.0 / (1.0 + jnp.exp(-(vk + vq)))
                msg_v[e, pl.ds(t * 16, 16)] = eta * vv
            return cc

        lax.fori_loop(0, C, _edge, 0)
        pltpu.sync_copy(msg_v, agg_sh.at[dst_v.at[j]], add=True)

    def _super(sup, carry):
        pltpu.sync_copy(src_hbm.at[wid, sup], src_v)
        pltpu.sync_copy(dst_hbm.at[wid, sup], dst_v)
        _start(0, k_v0, qv_v0, sem0)

        def _pair(jj, carry2):
            j = 2 * jj
            _start(j + 1, k_v1, qv_v1, sem1)
            _drain(j, k_v0, qv_v0, sem0)
            _compute_scatter(j, k_v0, qv_v0)

            @pl.when(j + 2 < SUP)
            def _():
                _start(j + 2, k_v0, qv_v0, sem0)

            _drain(j + 1, k_v1, qv_v1, sem1)
            _compute_scatter(j + 1, k_v1, qv_v1)
            return carry2

        lax.fori_loop(0, SUP // 2, _pair, 0)
        return carry

    lax.fori_loop(0, NSUP, _super, 0)
    plsc.subcore_barrier()

    for r in range((NCHUNK + NS - 1) // NS):
        idx = r * NS + s

        @pl.when(idx < NCHUNK)
        def _():
            pltpu.sync_copy(agg_sh.at[pl.ds(idx * ROWCHUNK, ROWCHUNK)],
                            out_hbm.at[c, pl.ds(idx * ROWCHUNK, ROWCHUNK)])


@functools.partial(
    pl.kernel,
    out_type=jax.ShapeDtypeStruct((NC, N, H), jnp.float32),
    mesh=plsc.VectorSubcoreMesh(
        core_axis_name="c", subcore_axis_name="s",
        num_cores=NC, num_subcores=NS),
    scratch_types=[
        pltpu.VMEM((SUP, C), jnp.int32),
        pltpu.VMEM((SUP, C), jnp.int32),
        pltpu.VMEM((C, H), jnp.float32),
        pltpu.VMEM((C, 2 * H), jnp.float32),
        pltpu.VMEM((C, H), jnp.float32),
        pltpu.VMEM((C, 2 * H), jnp.float32),
        pltpu.VMEM((C, H), jnp.float32),
        pltpu.VMEM_SHARED((N, H), jnp.float32),
        pltpu.SemaphoreType.DMA,
        pltpu.SemaphoreType.DMA,
    ],
)
def _edge_agg(k_hbm, qv_hbm, src_hbm, dst_hbm, zeros_hbm, out_hbm,
              src_v, dst_v, k_v0, qv_v0, k_v1, qv_v1, msg_v, agg_sh,
              sem0, sem1):
    _edge_body(k_hbm, qv_hbm, src_hbm, dst_hbm, zeros_hbm, out_hbm,
               src_v, dst_v, k_v0, qv_v0, k_v1, qv_v1, msg_v, agg_sh,
               sem0, sem1)


def _layer(x, src4, dst4, zeros, Wk, bk, Wq, bq, Wv, bv, Ws, b, lW, lb):
    wcat = jnp.concatenate([Wk, Wq, Wv, Ws], axis=1)
    bcat = jnp.concatenate([bk, bq, bv, b], axis=0)
    k, qv, skip = _qkvs(x, wcat, bcat)
    agg2 = _edge_agg(k, qv, src4, dst4, zeros)
    return _combine(agg2, skip, lW, lb)


def kernel(x, edge_index, c1_Wk, c1_bk, c1_Wq, c1_bq, c1_Wv, c1_bv, c1_Ws,
           c1_b, l1_W, l1_b, c2_Wk, c2_bk, c2_Wq, c2_bq, c2_Wv, c2_bv,
           c2_Ws, c2_b, l2_W, l2_b):
    src4 = edge_index[0].reshape(NW, NSUP, SUP, C)
    dst4 = edge_index[1].reshape(NW, NSUP, SUP, C)
    zeros = jnp.zeros((N, H), jnp.float32)
    h = _layer(x, src4, dst4, zeros, c1_Wk, c1_bk, c1_Wq, c1_bq, c1_Wv,
               c1_bv, c1_Ws, c1_b, l1_W, l1_b)
    h = _layer(h, src4, dst4, zeros, c2_Wk, c2_bk, c2_Wq, c2_bq, c2_Wv,
               c2_bv, c2_Ws, c2_b, l2_W, l2_b)
    return h


# speedup vs baseline: 8.0182x; 4.7017x over previous
import functools

import jax
import jax.numpy as jnp
import numpy as np
from jax import lax
from jax.experimental import pallas as pl
from jax.experimental.pallas import tpu as pltpu
from jax.experimental.pallas import tpu_sc as plsc

N = 10000
E = 320000
H = 128

NC = 2
NS = 16
NW = NC * NS
EPT = E // NW
C = 40
SUP = 50
NSUP = EPT // (C * SUP)
ROWCHUNK = 200
NCHUNK = N // ROWCHUNK

_PERM = np.empty((H,), np.int64)
for _t in range(H // 32):
    for _i in range(16):
        _PERM[32 * _t + 2 * _i] = 32 * _t + _i
        _PERM[32 * _t + 2 * _i + 1] = 32 * _t + 16 + _i


_MM_B = 1000


def _qkvs_body(x_ref, w_ref, b_ref, k_ref, qv_ref, s_ref):
    acc = jnp.dot(x_ref[...], w_ref[...], preferred_element_type=jnp.float32)
    acc = acc + b_ref[...]
    k_ref[...] = acc[:, :H]
    qv_ref[...] = acc[:, H:3 * H].astype(jnp.bfloat16)
    s_ref[...] = acc[:, 3 * H:]


def _qkvs(x, wcat, bcat):
    grid = (N // _MM_B,)
    return pl.pallas_call(
        _qkvs_body,
        grid=grid,
        in_specs=[
            pl.BlockSpec((_MM_B, H), lambda i: (i, 0)),
            pl.BlockSpec((H, 4 * H), lambda i: (0, 0)),
            pl.BlockSpec((4 * H,), lambda i: (0,)),
        ],
        out_specs=[
            pl.BlockSpec((_MM_B, H), lambda i: (i, 0)),
            pl.BlockSpec((_MM_B, 2 * H), lambda i: (i, 0)),
            pl.BlockSpec((_MM_B, H), lambda i: (i, 0)),
        ],
        out_shape=[
            jax.ShapeDtypeStruct((N, H), jnp.float32),
            jax.ShapeDtypeStruct((N, 2 * H), jnp.bfloat16),
            jax.ShapeDtypeStruct((N, H), jnp.float32),
        ],
    )(x, wcat, bcat)


def _combine_body(a_ref, s_ref, w_ref, b_ref, o_ref):
    t = a_ref[0] + a_ref[1] + s_ref[...]
    acc = jnp.dot(t, w_ref[...], preferred_element_type=jnp.float32)
    o_ref[...] = jnp.maximum(acc + b_ref[...], 0.0)


def _combine(agg2, skip, w, b):
    grid = (N // _MM_B,)
    return pl.pallas_call(
        _combine_body,
        grid=grid,
        in_specs=[
            pl.BlockSpec((NC, _MM_B, H), lambda i: (0, i, 0)),
            pl.BlockSpec((_MM_B, H), lambda i: (i, 0)),
            pl.BlockSpec((H, H), lambda i: (0, 0)),
            pl.BlockSpec((H,), lambda i: (0,)),
        ],
        out_specs=pl.BlockSpec((_MM_B, H), lambda i: (i, 0)),
        out_shape=jax.ShapeDtypeStruct((N, H), jnp.float32),
    )(agg2, skip, w, b)


def _edge_body(k_hbm, qv_hbm, src_hbm, dst_hbm, zeros_hbm, out_hbm,
               src_v, dst_v, k_v0, qv_v0, k_v1, qv_v1, msg_v0, msg_v1,
               agg_sh, sem0, sem1, sems0, sems1):
    c = lax.axis_index("c")
    s = lax.axis_index("s")
    wid = c * NS + s

    for r in range((NCHUNK + NS - 1) // NS):
        idx = r * NS + s

        @pl.when(idx < NCHUNK)
        def _():
            pltpu.sync_copy(zeros_hbm.at[pl.ds(idx * ROWCHUNK, ROWCHUNK)],
                            agg_sh.at[pl.ds(idx * ROWCHUNK, ROWCHUNK)])

    plsc.subcore_barrier()

    def _start(j, kbuf, qbuf, sem):
        pltpu.async_copy(k_hbm.at[dst_v.at[j]], kbuf, sem)
        pltpu.async_copy(qv_hbm.at[src_v.at[j]], qbuf, sem)

    def _drain(j, kbuf, qbuf, sem):
        pltpu.make_async_copy(k_hbm.at[dst_v.at[j]], kbuf, sem).wait()
        pltpu.make_async_copy(qv_hbm.at[src_v.at[j]], qbuf, sem).wait()

    himask = jnp.int32(-65536)

    def _unpk(w):
        lo = lax.bitcast_convert_type(lax.shift_left(w, 16), jnp.float32)
        hi = lax.bitcast_convert_type(lax.bitwise_and(w, himask),
                                      jnp.float32)
        return lo, hi

    def _compute(kbuf, qbuf, mbuf):
        def _edge(e, cc):
            for t in range(H // 32):
                nka = kbuf[e, pl.ds(t * 32, 16)]
                nkb = kbuf[e, pl.ds(t * 32 + 16, 16)]
                qa, qb = _unpk(qbuf[e, pl.ds(t * 16, 16)])
                va, vb = _unpk(qbuf[e, pl.ds(H // 2 + t * 16, 16)])
                mbuf[e, pl.ds(t * 32, 16)] = va / (1.0 + jnp.exp(nka - qa))
                mbuf[e, pl.ds(t * 32 + 16, 16)] = vb / (1.0 + jnp.exp(nkb - qb))
            return cc

        lax.fori_loop(0, C, _edge, 0)

    def _start_scatter(j, mbuf, sem):
        pltpu.async_copy(mbuf, agg_sh.at[dst_v.at[j]], sem, add=True)

    def _wait_scatter(mbuf, sem):
        pltpu.make_async_copy(mbuf, agg_sh.at[dst_v.at[0]], sem).wait()

    def _super(sup, carry):
        pltpu.sync_copy(src_hbm.at[wid, sup], src_v)
        pltpu.sync_copy(dst_hbm.at[wid, sup], dst_v)
        _start(0, k_v0, qv_v0, sem0)

        def _pair(jj, carry2):
            j = 2 * jj
            _start(j + 1, k_v1, qv_v1, sem1)
            _drain(j, k_v0, qv_v0, sem0)

            @pl.when(jj > 0)
            def _():
                _wait_scatter(msg_v0, sems0)

            _compute(k_v0, qv_v0, msg_v0)
            _start_scatter(j, msg_v0, sems0)

            @pl.when(j + 2 < SUP)
            def _():
                _start(j + 2, k_v0, qv_v0, sem0)

            _drain(j + 1, k_v1, qv_v1, sem1)

            @pl.when(jj > 0)
            def _():
                _wait_scatter(msg_v1, sems1)

            _compute(k_v1, qv_v1, msg_v1)
            _start_scatter(j + 1, msg_v1, sems1)
            return carry2

        lax.fori_loop(0, SUP // 2, _pair, 0)
        _wait_scatter(msg_v0, sems0)
        _wait_scatter(msg_v1, sems1)
        return carry

    lax.fori_loop(0, NSUP, _super, 0)
    plsc.subcore_barrier()

    for r in range((NCHUNK + NS - 1) // NS):
        idx = r * NS + s

        @pl.when(idx < NCHUNK)
        def _():
            pltpu.sync_copy(agg_sh.at[pl.ds(idx * ROWCHUNK, ROWCHUNK)],
                            out_hbm.at[c, pl.ds(idx * ROWCHUNK, ROWCHUNK)])


@functools.partial(
    pl.kernel,
    out_type=jax.ShapeDtypeStruct((NC, N, H), jnp.float32),
    mesh=plsc.VectorSubcoreMesh(
        core_axis_name="c", subcore_axis_name="s",
        num_cores=NC, num_subcores=NS),
    scratch_types=[
        pltpu.VMEM((SUP, C), jnp.int32),
        pltpu.VMEM((SUP, C), jnp.int32),
        pltpu.VMEM((C, H), jnp.float32),
        pltpu.VMEM((C, H), jnp.int32),
        pltpu.VMEM((C, H), jnp.float32),
        pltpu.VMEM((C, H), jnp.int32),
        pltpu.VMEM((C, H), jnp.float32),
        pltpu.VMEM((C, H), jnp.float32),
        pltpu.VMEM_SHARED((N, H), jnp.float32),
        pltpu.SemaphoreType.DMA,
        pltpu.SemaphoreType.DMA,
        pltpu.SemaphoreType.DMA,
        pltpu.SemaphoreType.DMA,
    ],
)
def _edge_agg(k_hbm, qv_hbm, src_hbm, dst_hbm, zeros_hbm, out_hbm,
              src_v, dst_v, k_v0, qv_v0, k_v1, qv_v1, msg_v0, msg_v1,
              agg_sh, sem0, sem1, sems0, sems1):
    _edge_body(k_hbm, qv_hbm, src_hbm, dst_hbm, zeros_hbm, out_hbm,
               src_v, dst_v, k_v0, qv_v0, k_v1, qv_v1, msg_v0, msg_v1,
               agg_sh, sem0, sem1, sems0, sems1)


def _layer(x, src4, dst4, zeros, Wk, bk, Wq, bq, Wv, bv, Ws, b, lW, lb):
    wcat = jnp.concatenate([-Wk, Wq[:, _PERM], Wv[:, _PERM], Ws], axis=1)
    bcat = jnp.concatenate([-bk, bq[_PERM], bv[_PERM], b], axis=0)
    k, qv, skip = _qkvs(x, wcat, bcat)
    qv_packed = lax.bitcast_convert_type(qv.reshape(N, H, 2), jnp.int32)
    agg2 = _edge_agg(k, qv_packed, src4, dst4, zeros)
    return _combine(agg2, skip, lW, lb)


def kernel(x, edge_index, c1_Wk, c1_bk, c1_Wq, c1_bq, c1_Wv, c1_bv, c1_Ws,
           c1_b, l1_W, l1_b, c2_Wk, c2_bk, c2_Wq, c2_bq, c2_Wv, c2_bv,
           c2_Ws, c2_b, l2_W, l2_b):
    src4 = edge_index[0].reshape(NW, NSUP, SUP, C)
    dst4 = edge_index[1].reshape(NW, NSUP, SUP, C)
    zeros = jnp.zeros((N, H), jnp.float32)
    h = _layer(x, src4, dst4, zeros, c1_Wk, c1_bk, c1_Wq, c1_bq, c1_Wv,
               c1_bv, c1_Ws, c1_b, l1_W, l1_b)
    h = _layer(h, src4, dst4, zeros, c2_Wk, c2_bk, c2_Wq, c2_bq, c2_Wv,
               c2_bv, c2_Ws, c2_b, l2_W, l2_b)
    return h
